# baseline (speedup 1.0000x reference)
import functools

import jax
import jax.numpy as jnp
from jax import lax
from jax.experimental import pallas as pl
from jax.experimental.pallas import tpu as pltpu
from jax.experimental.pallas import tpu_sc as plsc

VOCAB = 1000100
DIM = 64
LANES = 16
NUM_CORES = 2
NUM_SUBCORES = 16
NW = NUM_CORES * NUM_SUBCORES

CHUNK = 128
NBUF = 4


def _body(ids_hbm, w_hbm, out_hbm, idx_v, rows, gsems, osems,
          *, chunks_per_w):
    rows = list(rows)
    gsems = list(gsems)
    osems = list(osems)
    wid = lax.axis_index("s") * NUM_CORES + lax.axis_index("c")
    base = wid * chunks_per_w

    pltpu.sync_copy(ids_hbm.at[pl.ds(base, chunks_per_w)], idx_v)

    def clamp_chunk(c, _):
        for j in range(CHUNK // LANES):
            sl = pl.ds(j * LANES, LANES)
            idx_v[c, sl] = jnp.minimum(idx_v[c, sl], VOCAB - 1)
        return 0

    lax.fori_loop(0, chunks_per_w, clamp_chunk, 0, unroll=False)

    def start_gather(c, b):
        pltpu.async_copy(w_hbm.at[idx_v.at[c]], rows[b], gsems[b])

    def wait_gather(b):
        pltpu.make_async_copy(out_hbm.at[0], rows[b], gsems[b]).wait()

    def start_out(c, b):
        pltpu.async_copy(rows[b], out_hbm.at[base + c], osems[b])

    def wait_out(b):
        pltpu.make_async_copy(rows[b], out_hbm.at[0], osems[b]).wait()

    LAG = 2
    n_groups = (chunks_per_w + LAG) // NBUF + 1

    def group(g, _):
        for b in range(NBUF):
            c = g * NBUF + b

            @pl.when(c < chunks_per_w)
            def _gather():
                @pl.when(c >= NBUF)
                def _drain():
                    wait_out(b)

                start_gather(c, b)

            j = c - LAG
            bj = (b + NBUF - LAG) % NBUF

            @pl.when(jnp.logical_and(j >= 0, j < chunks_per_w))
            def _out():
                wait_gather(bj)
                start_out(j, bj)

        return 0

    lax.fori_loop(0, n_groups, group, 0, unroll=False)

    for b in range(NBUF):
        wait_out(b)


def kernel(input_ids, weight):
    batch, hist = input_ids.shape
    n = batch * hist
    assert n % (NW * CHUNK) == 0
    chunks_per_w = n // (NW * CHUNK)

    ids = input_ids.astype(jnp.int32).reshape(n // CHUNK, CHUNK)

    mesh = plsc.VectorSubcoreMesh(
        core_axis_name="c", subcore_axis_name="s",
        num_cores=NUM_CORES, num_subcores=NUM_SUBCORES)

    scratch = (
        [pltpu.VMEM((chunks_per_w, CHUNK), jnp.int32)]
        + [pltpu.VMEM((CHUNK, DIM), jnp.float32) for _ in range(NBUF)]
        + [pltpu.SemaphoreType.DMA for _ in range(2 * NBUF)]
    )

    def body(ids_hbm, w_hbm, out_hbm, *scr):
        _body(ids_hbm, w_hbm, out_hbm,
              scr[0], scr[1:1 + NBUF], scr[1 + NBUF:1 + 2 * NBUF],
              scr[1 + 2 * NBUF:], chunks_per_w=chunks_per_w)

    out = pl.kernel(
        body,
        out_type=jax.ShapeDtypeStruct((n // CHUNK, CHUNK, DIM), jnp.float32),
        mesh=mesh,
        scratch_types=scratch,
        compiler_params=pltpu.CompilerParams(use_tc_tiling_on_sc=False),
    )(ids, weight)
    return out.reshape(batch, hist, DIM)
